# baseline (speedup 1.0000x reference)
import functools

import jax
import jax.numpy as jnp
from jax.experimental import pallas as pl

DIM = 256
NUM_LAYERS = 4
SE = 0.1
SC_ = 0.1
SN = 0.05
BLOCK_ROWS = 2048


def _collapse_block(h_ref, w1_ref, b1_ref, w2_ref, b2_ref, ae_ref, ac_ref,
                    an_ref, out_ref):
    h = h_ref[...]
    b1 = b1_ref[...]
    b2 = b2_ref[...]

    def _norm_rows(x):
        n = jnp.sqrt(jnp.sum(x * x, axis=-1, keepdims=True))
        return x / jnp.maximum(n, 1e-12)

    e_dir = _norm_rows(ae_ref[...])
    c_dir = _norm_rows(ac_ref[...])
    n_dir = _norm_rows(an_ref[...])

    for _ in range(NUM_LAYERS):
        h_n = _norm_rows(h)
        a_e = jnp.sum(h_n * e_dir, axis=-1, keepdims=True)
        a_c = jnp.sum(h_n * c_dir, axis=-1, keepdims=True)
        a_n = jnp.sum(h_n * n_dir, axis=-1, keepdims=True)
        t = jnp.tanh(
            jax.lax.dot_general(h.astype(jnp.bfloat16), w1_ref[...],
                                (((1,), (1,)), ((), ())),
                                preferred_element_type=jnp.float32) + b1)
        delta = jax.lax.dot_general(t.astype(jnp.bfloat16), w2_ref[...],
                                    (((1,), (1,)), ((), ())),
                                    preferred_element_type=jnp.float32) + b2
        e_vec = _norm_rows(h - e_dir)
        c_vec = _norm_rows(h - c_dir)
        n_vec = _norm_rows(h - n_dir)
        h = (h + delta
             - SE * (0.38 - a_e) * e_vec
             - SC_ * (0.38 - a_c) * c_vec
             - SN * (0.38 - a_n) * n_vec)
        h_norm = jnp.sqrt(jnp.sum(h * h, axis=-1, keepdims=True))
        h = jnp.where(h_norm > 10.0, h * (10.0 / (h_norm + 1e-08)), h)
    out_ref[...] = h


@jax.jit
def kernel(h0, W1, b1, W2, b2, anchor_e, anchor_c, anchor_n):
    rows = h0.shape[0]
    grid = (rows // BLOCK_ROWS,)
    row_spec = pl.BlockSpec((BLOCK_ROWS, DIM), lambda i: (i, 0))
    full = pl.BlockSpec((DIM, DIM), lambda i: (0, 0))
    vec = pl.BlockSpec((1, DIM), lambda i: (0, 0))
    return pl.pallas_call(
        _collapse_block,
        grid=grid,
        in_specs=[row_spec, full, vec, full, vec, vec, vec, vec],
        out_specs=row_spec,
        out_shape=jax.ShapeDtypeStruct((rows, DIM), jnp.float32),
    )(h0, W1.astype(jnp.bfloat16), b1.reshape(1, DIM),
      W2.astype(jnp.bfloat16), b2.reshape(1, DIM),
      anchor_e.reshape(1, DIM), anchor_c.reshape(1, DIM),
      anchor_n.reshape(1, DIM))


# speedup vs baseline: 1.3852x; 1.3852x over previous
import jax
import jax.numpy as jnp
from jax.experimental import pallas as pl

DIM = 256
NUM_LAYERS = 4
SE = 0.1
SC_ = 0.1
SN = 0.05
BLOCK_ROWS = 2048


def _collapse_block(h_ref, w1_ref, b1_ref, w2_ref, b2_ref, ae_ref, ac_ref,
                    an_ref, out_ref):
    h = h_ref[...]
    b1 = b1_ref[...]
    b2 = b2_ref[...]

    def _unit(x):
        n = jnp.sqrt(jnp.sum(x * x, axis=-1, keepdims=True))
        return x / jnp.maximum(n, 1e-12)

    e_dir = _unit(ae_ref[...])
    c_dir = _unit(ac_ref[...])
    n_dir = _unit(an_ref[...])

    hh = jnp.sum(h * h, axis=-1, keepdims=True)
    for _ in range(NUM_LAYERS):
        inv_hn = 1.0 / jnp.maximum(jnp.sqrt(hh), 1e-12)
        he = jnp.sum(h * e_dir, axis=-1, keepdims=True)
        hc = jnp.sum(h * c_dir, axis=-1, keepdims=True)
        hn = jnp.sum(h * n_dir, axis=-1, keepdims=True)
        ce = SE * (0.38 - he * inv_hn) * jax.lax.rsqrt(
            jnp.maximum(hh - 2.0 * he + 1.0, 1e-24))
        cc = SC_ * (0.38 - hc * inv_hn) * jax.lax.rsqrt(
            jnp.maximum(hh - 2.0 * hc + 1.0, 1e-24))
        cn = SN * (0.38 - hn * inv_hn) * jax.lax.rsqrt(
            jnp.maximum(hh - 2.0 * hn + 1.0, 1e-24))
        t = jnp.tanh(
            jax.lax.dot_general(h, w1_ref[...], (((1,), (1,)), ((), ())),
                                preferred_element_type=jnp.float32) + b1)
        delta = jax.lax.dot_general(t, w2_ref[...], (((1,), (1,)), ((), ())),
                                    preferred_element_type=jnp.float32) + b2
        h = (h * (1.0 - ce - cc - cn) + delta
             + ce * e_dir + cc * c_dir + cn * n_dir)
        hh = jnp.sum(h * h, axis=-1, keepdims=True)
        norm = jnp.sqrt(hh)
        scale = jnp.where(norm > 10.0, 10.0 / (norm + 1e-08), 1.0)
        h = h * scale
        hh = hh * scale * scale
    out_ref[...] = h


@jax.jit
def kernel(h0, W1, b1, W2, b2, anchor_e, anchor_c, anchor_n):
    rows = h0.shape[0]
    grid = (rows // BLOCK_ROWS,)
    row_spec = pl.BlockSpec((BLOCK_ROWS, DIM), lambda i: (i, 0))
    full = pl.BlockSpec((DIM, DIM), lambda i: (0, 0))
    vec = pl.BlockSpec((1, DIM), lambda i: (0, 0))
    return pl.pallas_call(
        _collapse_block,
        grid=grid,
        in_specs=[row_spec, full, vec, full, vec, vec, vec, vec],
        out_specs=row_spec,
        out_shape=jax.ShapeDtypeStruct((rows, DIM), jnp.float32),
    )(h0, W1, b1.reshape(1, DIM), W2, b2.reshape(1, DIM),
      anchor_e.reshape(1, DIM), anchor_c.reshape(1, DIM),
      anchor_n.reshape(1, DIM))
